# baseline (speedup 1.0000x reference)
import functools

import jax
import jax.numpy as jnp
from jax import lax
from jax.experimental import pallas as pl
from jax.experimental.pallas import tpu as pltpu
from jax.experimental.pallas import tpu_sc as plsc

D = 128
H = 8
DH = D // H
EPS = 1e-5

NBLK = 1000
EBLK = 2000


def _ln(x, g, b):
    m = jnp.mean(x, axis=-1, keepdims=True)
    v = jnp.mean((x - m) ** 2, axis=-1, keepdims=True)
    return (x - m) / jnp.sqrt(v + EPS) * g + b


def _head_mask():
    d = lax.broadcasted_iota(jnp.int32, (D, 16), 0)
    t = lax.broadcasted_iota(jnp.int32, (D, 16), 1)
    return (d // DH == t // 2).astype(jnp.float32)


def _rpe_body(eattr_ref, w_ref, vecs_ref, out_ref):
    h = jnp.dot(eattr_ref[...], w_ref[...], preferred_element_type=jnp.float32)
    h = h + vecs_ref[0:1, :]
    out_ref[...] = jax.nn.relu(
        _ln(h, vecs_ref[1:2, :], vecs_ref[2:3, :])).astype(jnp.bfloat16)


def _rpe_call(eattr_pad, w_pad, vecs, E):
    grid = E // EBLK
    return pl.pallas_call(
        _rpe_body,
        grid=(grid,),
        in_specs=[
            pl.BlockSpec((EBLK, 16), lambda i: (i, 0)),
            pl.BlockSpec((16, D), lambda i: (0, 0)),
            pl.BlockSpec((3, D), lambda i: (0, 0)),
        ],
        out_specs=pl.BlockSpec((EBLK, D), lambda i: (i, 0)),
        out_shape=jax.ShapeDtypeStruct((E, D), jnp.bfloat16),
    )(eattr_pad, w_pad, vecs)


def _node_pre_body(x_ref, wd_ref, ws_ref, td_ref, ts_ref):
    x = x_ref[...]
    td = jnp.dot(x, wd_ref[...], preferred_element_type=jnp.float32)
    a = lax.bitcast_convert_type(td[:, :D].astype(jnp.bfloat16).astype(jnp.float32),
                                 jnp.int32)
    qq = lax.bitcast_convert_type(td[:, D:].astype(jnp.bfloat16).astype(jnp.float32),
                                  jnp.int32)
    td_ref[...] = lax.shift_right_logical(a, 16) | qq
    ts_ref[...] = jnp.dot(x, ws_ref[...], preferred_element_type=jnp.float32)


def _node_pre_call(x, w_dst, w_src, N):
    grid = N // NBLK
    return pl.pallas_call(
        _node_pre_body,
        grid=(grid,),
        in_specs=[
            pl.BlockSpec((NBLK, D), lambda i: (i, 0)),
            pl.BlockSpec((D, 2 * D), lambda i: (0, 0)),
            pl.BlockSpec((D, D), lambda i: (0, 0)),
        ],
        out_specs=[
            pl.BlockSpec((NBLK, D), lambda i: (i, 0)),
            pl.BlockSpec((NBLK, D), lambda i: (i, 0)),
        ],
        out_shape=[
            jax.ShapeDtypeStruct((N, D), jnp.int32),
            jax.ShapeDtypeStruct((N, D), jnp.float32),
        ],
    )(x, w_dst, w_src)


def _edge_body(g1_ref, g2_ref, ea_ref, w3_ref, wkv_ref, vecs_ref,
               ne_ref, wv_ref, we_ref):
    ea = ea_ref[...].astype(jnp.float32)
    raw = g1_ref[...]
    g1a = lax.bitcast_convert_type(lax.shift_left(raw, 16), jnp.float32)
    q = lax.bitcast_convert_type(raw & jnp.int32(-65536), jnp.float32)
    vec = lambda i: vecs_ref[i:i + 1, :]
    mem_pre = (g1a + g2_ref[...]
               + jnp.dot(ea, w3_ref[...], preferred_element_type=jnp.float32)
               + vec(0))
    mem = jax.nn.relu(_ln(mem_pre, vec(1), vec(2)))
    tmp = jnp.dot(mem, wkv_ref[...], preferred_element_type=jnp.float32)
    delta = jax.nn.relu(_ln(tmp[:, :D] + vec(3), vec(4), vec(5)))
    ne_ref[...] = _ln(ea + delta, vec(6), vec(7)).astype(jnp.bfloat16)
    k = tmp[:, D:2 * D]
    v = tmp[:, 2 * D:3 * D]
    m16 = _head_mask()
    logits16 = jnp.dot(q * k, m16, preferred_element_type=jnp.float32) * (1.0 / (DH ** 0.5))
    e16 = jnp.exp(logits16)
    e128 = jnp.dot(e16, 0.5 * m16.T, preferred_element_type=jnp.float32)
    wv_ref[...] = e128 * v
    we_ref[...] = jnp.concatenate(
        [e16, jnp.zeros((e16.shape[0], D - 16), jnp.float32)], axis=1)


def _edge_call(g1, g2, ea, w3, wkv, vecs, E):
    grid = E // EBLK
    return pl.pallas_call(
        _edge_body,
        grid=(grid,),
        in_specs=[
            pl.BlockSpec((EBLK, D), lambda i: (i, 0)),
            pl.BlockSpec((EBLK, D), lambda i: (i, 0)),
            pl.BlockSpec((EBLK, D), lambda i: (i, 0)),
            pl.BlockSpec((D, D), lambda i: (0, 0)),
            pl.BlockSpec((D, 3 * D), lambda i: (0, 0)),
            pl.BlockSpec((8, D), lambda i: (0, 0)),
        ],
        out_specs=[
            pl.BlockSpec((EBLK, D), lambda i: (i, 0)),
            pl.BlockSpec((EBLK, D), lambda i: (i, 0)),
            pl.BlockSpec((EBLK, D), lambda i: (i, 0)),
        ],
        out_shape=[
            jax.ShapeDtypeStruct((E, D), jnp.bfloat16),
            jax.ShapeDtypeStruct((E, D), jnp.float32),
            jax.ShapeDtypeStruct((E, D), jnp.float32),
        ],
    )(g1, g2, ea, w3, wkv, vecs)


def _node_post_body(x_ref, pv0_ref, pv1_ref, pe0_ref, pe1_ref,
                    wo_ref, w1_ref, w2_ref, vecs_ref, fb1_ref, out_ref):
    vec = lambda i: vecs_ref[i:i + 1, :]
    num = pv0_ref[...] + pv1_ref[...]
    den16 = pe0_ref[...][:, :16] + pe1_ref[...][:, :16]
    m16 = _head_mask()
    den128 = jnp.dot(den16, 0.5 * m16.T, preferred_element_type=jnp.float32)
    u = num / (den128 + 1e-16)
    aggr = jnp.dot(u, wo_ref[...], preferred_element_type=jnp.float32)
    x1 = _ln(x_ref[...] + aggr, vec(0), vec(1))
    h = jax.nn.relu(jnp.dot(x1, w1_ref[...], preferred_element_type=jnp.float32)
                    + fb1_ref[0:1, :])
    h = jnp.dot(h, w2_ref[...], preferred_element_type=jnp.float32) + vec(2)
    out_ref[...] = _ln(x1 + h, vec(3), vec(4))


def _node_post_call(x, pv0, pv1, pe0, pe1, wo, w1, w2, vecs, fb1, N):
    grid = N // NBLK
    return pl.pallas_call(
        _node_post_body,
        grid=(grid,),
        in_specs=[
            pl.BlockSpec((NBLK, D), lambda i: (i, 0)),
            pl.BlockSpec((NBLK, D), lambda i: (i, 0)),
            pl.BlockSpec((NBLK, D), lambda i: (i, 0)),
            pl.BlockSpec((NBLK, D), lambda i: (i, 0)),
            pl.BlockSpec((NBLK, D), lambda i: (i, 0)),
            pl.BlockSpec((D, D), lambda i: (0, 0)),
            pl.BlockSpec((D, 2 * D), lambda i: (0, 0)),
            pl.BlockSpec((2 * D, D), lambda i: (0, 0)),
            pl.BlockSpec((8, D), lambda i: (0, 0)),
            pl.BlockSpec((8, 2 * D), lambda i: (0, 0)),
        ],
        out_specs=pl.BlockSpec((NBLK, D), lambda i: (i, 0)),
        out_shape=jax.ShapeDtypeStruct((N, D), jnp.float32),
    )(x, pv0, pv1, pe0, pe1, wo, w1, w2, vecs, fb1)


_NC, _NS = 2, 16
_NW = _NC * _NS
_GCH = 128
_SC_MESH = plsc.VectorSubcoreMesh(core_axis_name="c", subcore_axis_name="s")


def _gather(t_dst, t_src, dst, src, N, E):
    nchunks = E // _GCH
    per = -(-nchunks // _NW)

    def body(td_hbm, ts_hbm, dst_hbm, src_hbm, g1_hbm, g2_hbm,
             idx_d, idx_s, buf1, buf2, sem1, sem2):
        wid = lax.axis_index("s") * _NC + lax.axis_index("c")

        def step(i, carry):
            chunk = wid + _NW * i

            @pl.when(chunk < nchunks)
            def _():
                base = chunk * _GCH
                pltpu.sync_copy(dst_hbm.at[pl.ds(base, _GCH)], idx_d)
                pltpu.sync_copy(src_hbm.at[pl.ds(base, _GCH)], idx_s)
                cp1 = pltpu.async_copy(td_hbm.at[idx_d], buf1, sem1)
                cp2 = pltpu.async_copy(ts_hbm.at[idx_s], buf2, sem2)
                cp1.wait()
                cp2.wait()
                pltpu.sync_copy(buf1, g1_hbm.at[pl.ds(base, _GCH)])
                pltpu.sync_copy(buf2, g2_hbm.at[pl.ds(base, _GCH)])

            return carry

        lax.fori_loop(0, per, step, 0)

    call = pl.kernel(
        body,
        out_type=[
            jax.ShapeDtypeStruct((E, D), jnp.int32),
            jax.ShapeDtypeStruct((E, D), jnp.float32),
        ],
        mesh=_SC_MESH,
        scratch_types=[
            pltpu.VMEM((_GCH,), jnp.int32),
            pltpu.VMEM((_GCH,), jnp.int32),
            pltpu.VMEM((_GCH, D), jnp.int32),
            pltpu.VMEM((_GCH, D), jnp.float32),
            pltpu.SemaphoreType.DMA,
            pltpu.SemaphoreType.DMA,
        ],
    )
    return call(t_dst, t_src, dst, src)


def _scatter_add(w_edges, dst, zeros_slab, N, E):
    nchunks = E // _GCH
    per = -(-nchunks // _NW)
    W = D
    full = 640
    tail = N - 15 * full

    def body(w_hbm, dst_hbm, z_hbm, p_hbm, idx0, idx1, buf0, buf1, acc,
             lsem0, lsem1, ssem):
        c = lax.axis_index("c")
        s = lax.axis_index("s")
        wid = s * _NC + c

        @pl.when(s < _NS - 1)
        def _():
            pltpu.sync_copy(z_hbm, acc.at[pl.ds(s * full, full)])

        @pl.when(s == _NS - 1)
        def _():
            pltpu.sync_copy(z_hbm.at[pl.ds(0, tail)],
                            acc.at[pl.ds((_NS - 1) * full, tail)])

        plsc.subcore_barrier()

        def base_of(i):
            chunk = wid + _NW * i
            return jnp.minimum(chunk, nchunks - 1) * _GCH

        def load(i, idx_b, buf, lsem):
            b = base_of(i)
            pltpu.async_copy(dst_hbm.at[pl.ds(b, _GCH)], idx_b, lsem)
            pltpu.async_copy(w_hbm.at[pl.ds(b, _GCH)], buf, lsem)

        def wait_load(idx_b, buf, lsem):
            pltpu.make_async_copy(dst_hbm.at[pl.ds(0, _GCH)], idx_b, lsem).wait()
            pltpu.make_async_copy(w_hbm.at[pl.ds(0, _GCH)], buf, lsem).wait()

        def scat(i, idx_b, buf):
            @pl.when(wid + _NW * i < nchunks)
            def _():
                pltpu.async_copy(buf, acc.at[idx_b], ssem, add=True).wait()

        load(0, idx0, buf0, lsem0)

        def step(t, carry):
            wait_load(idx0, buf0, lsem0)
            load(2 * t + 1, idx1, buf1, lsem1)
            scat(2 * t, idx0, buf0)
            wait_load(idx1, buf1, lsem1)
            load(2 * t + 2, idx0, buf0, lsem0)
            scat(2 * t + 1, idx1, buf1)
            return carry

        lax.fori_loop(0, (per + 1) // 2, step, 0)
        wait_load(idx0, buf0, lsem0)
        plsc.subcore_barrier()

        @pl.when(s < _NS - 1)
        def _():
            pltpu.sync_copy(acc.at[pl.ds(s * full, full)],
                            p_hbm.at[c, pl.ds(s * full, full)])

        @pl.when(s == _NS - 1)
        def _():
            pltpu.sync_copy(acc.at[pl.ds((_NS - 1) * full, tail)],
                            p_hbm.at[c, pl.ds((_NS - 1) * full, tail)])

    call = pl.kernel(
        body,
        out_type=jax.ShapeDtypeStruct((_NC, N, W), jnp.float32),
        mesh=_SC_MESH,
        scratch_types=[
            pltpu.VMEM((_GCH,), jnp.int32),
            pltpu.VMEM((_GCH,), jnp.int32),
            pltpu.VMEM((_GCH, W), jnp.float32),
            pltpu.VMEM((_GCH, W), jnp.float32),
            pltpu.VMEM_SHARED((N, W), jnp.float32),
            pltpu.SemaphoreType.DMA,
            pltpu.SemaphoreType.DMA,
            pltpu.SemaphoreType.DMA,
        ],
    )
    p = call(w_edges, dst, zeros_slab)
    return p[0], p[1]


def kernel(lane_feats, edge_indexs, edge_attrs, params):
    N = lane_feats.shape[0]
    E = edge_attrs.shape[0]
    P = 4
    EP = E // P
    src = edge_indexs[0]
    dst = edge_indexs[1]
    srcs = [src[i * EP:(i + 1) * EP] for i in range(P)]
    dsts = [dst[i * EP:(i + 1) * EP] for i in range(P)]
    zeros_slab = jnp.zeros((640, D), jnp.float32)

    eattr_pad = jnp.pad(edge_attrs, ((0, 0), (0, 16 - edge_attrs.shape[1])))
    rpe_w_pad = jnp.pad(params['rpe_W'], ((0, 16 - params['rpe_W'].shape[0]), (0, 0)))
    rpe_vecs = jnp.stack([params['rpe_b'], params['rpe_ln_g'], params['rpe_ln_b']])
    eas = [_rpe_call(eattr_pad[i * EP:(i + 1) * EP], rpe_w_pad, rpe_vecs, EP)
           for i in range(P)]

    x = lane_feats
    for p in params['layers']:
        w_dst = jnp.concatenate([p['mem_W'][:D], p['Wq']], axis=1)
        w_src = p['mem_W'][D:2 * D]
        w3 = p['mem_W'][2 * D:]
        wkv = jnp.concatenate([p['eu_W'], p['Wk'], p['Wv']], axis=1)
        evecs = jnp.stack([p['mem_b'], p['mem_ln_g'], p['mem_ln_b'],
                           p['eu_b'], p['eu_ln_g'], p['eu_ln_b'],
                           p['en_g'], p['en_b']])
        nvecs = jnp.stack([p['n1_g'], p['n1_b'], p['ffn_b2'],
                           p['n2_g'], p['n2_b'],
                           p['n1_g'], p['n1_g'], p['n1_g']])
        fb1 = jnp.broadcast_to(p['ffn_b1'][None, :], (8, 2 * D))

        t_dst, t_src = _node_pre_call(x, w_dst, w_src, N)
        wvs, wes = [], []
        for i in range(P):
            g1, g2 = _gather(t_dst, t_src, dsts[i], srcs[i], N, EP)
            eas[i], wv, we = _edge_call(g1, g2, eas[i], w3, wkv, evecs, EP)
            wvs.append(wv)
            wes.append(we)
        pv0 = pv1 = pe0 = pe1 = None
        for i in range(P):
            a0, a1 = _scatter_add(wvs[i], dsts[i], zeros_slab, N, EP)
            b0, b1 = _scatter_add(wes[i], dsts[i], zeros_slab, N, EP)
            pv0 = a0 if pv0 is None else pv0 + a0
            pv1 = a1 if pv1 is None else pv1 + a1
            pe0 = b0 if pe0 is None else pe0 + b0
            pe1 = b1 if pe1 is None else pe1 + b1
        x = _node_post_call(x, pv0, pv1, pe0, pe1, p['Wo'], p['ffn_W1'],
                            p['ffn_W2'], nvecs, fb1, N)
    return x


# speedup vs baseline: 1.0257x; 1.0257x over previous
import functools

import jax
import jax.numpy as jnp
from jax import lax
from jax.experimental import pallas as pl
from jax.experimental.pallas import tpu as pltpu
from jax.experimental.pallas import tpu_sc as plsc

D = 128
H = 8
DH = D // H
EPS = 1e-5

NBLK = 1000
EBLK = 2000


def _ln(x, g, b):
    m = jnp.mean(x, axis=-1, keepdims=True)
    v = jnp.mean((x - m) ** 2, axis=-1, keepdims=True)
    return (x - m) / jnp.sqrt(v + EPS) * g + b


def _head_mask():
    d = lax.broadcasted_iota(jnp.int32, (D, 16), 0)
    t = lax.broadcasted_iota(jnp.int32, (D, 16), 1)
    return (d // DH == t // 2).astype(jnp.float32)


def _rpe_body(eattr_ref, w_ref, vecs_ref, out_ref):
    h = jnp.dot(eattr_ref[...], w_ref[...], preferred_element_type=jnp.float32)
    h = h + vecs_ref[0:1, :]
    out_ref[...] = jax.nn.relu(
        _ln(h, vecs_ref[1:2, :], vecs_ref[2:3, :])).astype(jnp.bfloat16)


def _rpe_call(eattr_pad, w_pad, vecs, E):
    grid = E // EBLK
    return pl.pallas_call(
        _rpe_body,
        grid=(grid,),
        in_specs=[
            pl.BlockSpec((EBLK, 16), lambda i: (i, 0)),
            pl.BlockSpec((16, D), lambda i: (0, 0)),
            pl.BlockSpec((3, D), lambda i: (0, 0)),
        ],
        out_specs=pl.BlockSpec((EBLK, D), lambda i: (i, 0)),
        out_shape=jax.ShapeDtypeStruct((E, D), jnp.bfloat16),
    )(eattr_pad, w_pad, vecs)


def _node_pre_body(x_ref, wd_ref, ws_ref, td_ref, ts_ref):
    x = x_ref[...]
    td = jnp.dot(x, wd_ref[...], preferred_element_type=jnp.float32)
    a = lax.bitcast_convert_type(td[:, :D].astype(jnp.bfloat16).astype(jnp.float32),
                                 jnp.int32)
    qq = lax.bitcast_convert_type(td[:, D:].astype(jnp.bfloat16).astype(jnp.float32),
                                  jnp.int32)
    td_ref[...] = lax.shift_right_logical(a, 16) | qq
    ts_ref[...] = jnp.dot(x, ws_ref[...], preferred_element_type=jnp.float32)


def _node_pre_call(x, w_dst, w_src, N):
    grid = N // NBLK
    return pl.pallas_call(
        _node_pre_body,
        grid=(grid,),
        in_specs=[
            pl.BlockSpec((NBLK, D), lambda i: (i, 0)),
            pl.BlockSpec((D, 2 * D), lambda i: (0, 0)),
            pl.BlockSpec((D, D), lambda i: (0, 0)),
        ],
        out_specs=[
            pl.BlockSpec((NBLK, D), lambda i: (i, 0)),
            pl.BlockSpec((NBLK, D), lambda i: (i, 0)),
        ],
        out_shape=[
            jax.ShapeDtypeStruct((N, D), jnp.int32),
            jax.ShapeDtypeStruct((N, D), jnp.float32),
        ],
    )(x, w_dst, w_src)


def _edge_body(g1_ref, g2_ref, ea_ref, w3_ref, wkv_ref, vecs_ref,
               ne_ref, wv_ref, we_ref):
    ea = ea_ref[...].astype(jnp.float32)
    raw = g1_ref[...]
    g1a = lax.bitcast_convert_type(lax.shift_left(raw, 16), jnp.float32)
    q = lax.bitcast_convert_type(raw & jnp.int32(-65536), jnp.float32)
    vec = lambda i: vecs_ref[i:i + 1, :]
    mem_pre = (g1a + g2_ref[...]
               + jnp.dot(ea, w3_ref[...], preferred_element_type=jnp.float32)
               + vec(0))
    mem = jax.nn.relu(_ln(mem_pre, vec(1), vec(2)))
    tmp = jnp.dot(mem, wkv_ref[...], preferred_element_type=jnp.float32)
    delta = jax.nn.relu(_ln(tmp[:, :D] + vec(3), vec(4), vec(5)))
    ne_ref[...] = _ln(ea + delta, vec(6), vec(7)).astype(jnp.bfloat16)
    k = tmp[:, D:2 * D]
    v = tmp[:, 2 * D:3 * D]
    m16 = _head_mask()
    logits16 = jnp.dot(q * k, m16, preferred_element_type=jnp.float32) * (1.0 / (DH ** 0.5))
    e16 = jnp.exp(logits16)
    e128 = jnp.dot(e16, 0.5 * m16.T, preferred_element_type=jnp.float32)
    wv_ref[...] = e128 * v
    we_ref[...] = jnp.concatenate(
        [e16, jnp.zeros((e16.shape[0], D - 16), jnp.float32)], axis=1)


def _edge_call(g1, g2, ea, w3, wkv, vecs, E):
    grid = E // EBLK
    return pl.pallas_call(
        _edge_body,
        grid=(grid,),
        in_specs=[
            pl.BlockSpec((EBLK, D), lambda i: (i, 0)),
            pl.BlockSpec((EBLK, D), lambda i: (i, 0)),
            pl.BlockSpec((EBLK, D), lambda i: (i, 0)),
            pl.BlockSpec((D, D), lambda i: (0, 0)),
            pl.BlockSpec((D, 3 * D), lambda i: (0, 0)),
            pl.BlockSpec((8, D), lambda i: (0, 0)),
        ],
        out_specs=[
            pl.BlockSpec((EBLK, D), lambda i: (i, 0)),
            pl.BlockSpec((EBLK, D), lambda i: (i, 0)),
            pl.BlockSpec((EBLK, D), lambda i: (i, 0)),
        ],
        out_shape=[
            jax.ShapeDtypeStruct((E, D), jnp.bfloat16),
            jax.ShapeDtypeStruct((E, D), jnp.float32),
            jax.ShapeDtypeStruct((E, D), jnp.float32),
        ],
    )(g1, g2, ea, w3, wkv, vecs)


def _node_post_body(x_ref, pv0_ref, pv1_ref, pe0_ref, pe1_ref,
                    wo_ref, w1_ref, w2_ref, vecs_ref, fb1_ref, out_ref):
    vec = lambda i: vecs_ref[i:i + 1, :]
    num = pv0_ref[...] + pv1_ref[...]
    den16 = pe0_ref[...][:, :16] + pe1_ref[...][:, :16]
    m16 = _head_mask()
    den128 = jnp.dot(den16, 0.5 * m16.T, preferred_element_type=jnp.float32)
    u = num / (den128 + 1e-16)
    aggr = jnp.dot(u, wo_ref[...], preferred_element_type=jnp.float32)
    x1 = _ln(x_ref[...] + aggr, vec(0), vec(1))
    h = jax.nn.relu(jnp.dot(x1, w1_ref[...], preferred_element_type=jnp.float32)
                    + fb1_ref[0:1, :])
    h = jnp.dot(h, w2_ref[...], preferred_element_type=jnp.float32) + vec(2)
    out_ref[...] = _ln(x1 + h, vec(3), vec(4))


def _node_post_call(x, pv0, pv1, pe0, pe1, wo, w1, w2, vecs, fb1, N):
    grid = N // NBLK
    return pl.pallas_call(
        _node_post_body,
        grid=(grid,),
        in_specs=[
            pl.BlockSpec((NBLK, D), lambda i: (i, 0)),
            pl.BlockSpec((NBLK, D), lambda i: (i, 0)),
            pl.BlockSpec((NBLK, D), lambda i: (i, 0)),
            pl.BlockSpec((NBLK, D), lambda i: (i, 0)),
            pl.BlockSpec((NBLK, D), lambda i: (i, 0)),
            pl.BlockSpec((D, D), lambda i: (0, 0)),
            pl.BlockSpec((D, 2 * D), lambda i: (0, 0)),
            pl.BlockSpec((2 * D, D), lambda i: (0, 0)),
            pl.BlockSpec((8, D), lambda i: (0, 0)),
            pl.BlockSpec((8, 2 * D), lambda i: (0, 0)),
        ],
        out_specs=pl.BlockSpec((NBLK, D), lambda i: (i, 0)),
        out_shape=jax.ShapeDtypeStruct((N, D), jnp.float32),
    )(x, pv0, pv1, pe0, pe1, wo, w1, w2, vecs, fb1)


_NC, _NS = 2, 16
_NW = _NC * _NS
_GCH = 128
_SC_MESH = plsc.VectorSubcoreMesh(core_axis_name="c", subcore_axis_name="s")


def _gather(t_dst, t_src, dst, src, N, E):
    nchunks = E // _GCH
    per = -(-nchunks // _NW)

    def body(td_hbm, ts_hbm, dst_hbm, src_hbm, g1_hbm, g2_hbm,
             idx_d, idx_s, buf1, buf2, sem1, sem2):
        wid = lax.axis_index("s") * _NC + lax.axis_index("c")

        def step(i, carry):
            chunk = wid + _NW * i

            @pl.when(chunk < nchunks)
            def _():
                base = chunk * _GCH
                pltpu.sync_copy(dst_hbm.at[pl.ds(base, _GCH)], idx_d)
                pltpu.sync_copy(src_hbm.at[pl.ds(base, _GCH)], idx_s)
                cp1 = pltpu.async_copy(td_hbm.at[idx_d], buf1, sem1)
                cp2 = pltpu.async_copy(ts_hbm.at[idx_s], buf2, sem2)
                cp1.wait()
                cp2.wait()
                pltpu.sync_copy(buf1, g1_hbm.at[pl.ds(base, _GCH)])
                pltpu.sync_copy(buf2, g2_hbm.at[pl.ds(base, _GCH)])

            return carry

        lax.fori_loop(0, per, step, 0)

    call = pl.kernel(
        body,
        out_type=[
            jax.ShapeDtypeStruct((E, D), jnp.int32),
            jax.ShapeDtypeStruct((E, D), jnp.float32),
        ],
        mesh=_SC_MESH,
        scratch_types=[
            pltpu.VMEM((_GCH,), jnp.int32),
            pltpu.VMEM((_GCH,), jnp.int32),
            pltpu.VMEM((_GCH, D), jnp.int32),
            pltpu.VMEM((_GCH, D), jnp.float32),
            pltpu.SemaphoreType.DMA,
            pltpu.SemaphoreType.DMA,
        ],
    )
    return call(t_dst, t_src, dst, src)


def _scatter_add(w_edges, dst, zeros_slab, N, E):
    nchunks = E // _GCH
    per = -(-nchunks // _NW)
    W = D
    full = 640
    tail = N - 15 * full

    def body(w_hbm, dst_hbm, z_hbm, p_hbm, idx0, idx1, buf0, buf1, acc,
             lsem0, lsem1, ssem):
        c = lax.axis_index("c")
        s = lax.axis_index("s")
        wid = s * _NC + c

        @pl.when(s < _NS - 1)
        def _():
            pltpu.sync_copy(z_hbm, acc.at[pl.ds(s * full, full)])

        @pl.when(s == _NS - 1)
        def _():
            pltpu.sync_copy(z_hbm.at[pl.ds(0, tail)],
                            acc.at[pl.ds((_NS - 1) * full, tail)])

        plsc.subcore_barrier()

        def base_of(i):
            chunk = wid + _NW * i
            return jnp.minimum(chunk, nchunks - 1) * _GCH

        def load(i, idx_b, buf, lsem):
            b = base_of(i)
            pltpu.async_copy(dst_hbm.at[pl.ds(b, _GCH)], idx_b, lsem)
            pltpu.async_copy(w_hbm.at[pl.ds(b, _GCH)], buf, lsem)

        def wait_load(idx_b, buf, lsem):
            pltpu.make_async_copy(dst_hbm.at[pl.ds(0, _GCH)], idx_b, lsem).wait()
            pltpu.make_async_copy(w_hbm.at[pl.ds(0, _GCH)], buf, lsem).wait()

        def scat(i, idx_b, buf):
            @pl.when(wid + _NW * i < nchunks)
            def _():
                pltpu.async_copy(buf, acc.at[idx_b], ssem, add=True).wait()

        load(0, idx0, buf0, lsem0)

        def step(t, carry):
            wait_load(idx0, buf0, lsem0)
            load(2 * t + 1, idx1, buf1, lsem1)
            scat(2 * t, idx0, buf0)
            wait_load(idx1, buf1, lsem1)
            load(2 * t + 2, idx0, buf0, lsem0)
            scat(2 * t + 1, idx1, buf1)
            return carry

        lax.fori_loop(0, (per + 1) // 2, step, 0)
        wait_load(idx0, buf0, lsem0)
        plsc.subcore_barrier()

        @pl.when(s < _NS - 1)
        def _():
            pltpu.sync_copy(acc.at[pl.ds(s * full, full)],
                            p_hbm.at[c, pl.ds(s * full, full)])

        @pl.when(s == _NS - 1)
        def _():
            pltpu.sync_copy(acc.at[pl.ds((_NS - 1) * full, tail)],
                            p_hbm.at[c, pl.ds((_NS - 1) * full, tail)])

    call = pl.kernel(
        body,
        out_type=jax.ShapeDtypeStruct((_NC, N, W), jnp.float32),
        mesh=_SC_MESH,
        scratch_types=[
            pltpu.VMEM((_GCH,), jnp.int32),
            pltpu.VMEM((_GCH,), jnp.int32),
            pltpu.VMEM((_GCH, W), jnp.float32),
            pltpu.VMEM((_GCH, W), jnp.float32),
            pltpu.VMEM_SHARED((N, W), jnp.float32),
            pltpu.SemaphoreType.DMA,
            pltpu.SemaphoreType.DMA,
            pltpu.SemaphoreType.DMA,
        ],
    )
    p = call(w_edges, dst, zeros_slab)
    return p[0], p[1]


def kernel(lane_feats, edge_indexs, edge_attrs, params):
    N = lane_feats.shape[0]
    E = edge_attrs.shape[0]
    P = 2
    EP = E // P
    src = edge_indexs[0]
    dst = edge_indexs[1]
    srcs = [src[i * EP:(i + 1) * EP] for i in range(P)]
    dsts = [dst[i * EP:(i + 1) * EP] for i in range(P)]
    zeros_slab = jnp.zeros((640, D), jnp.float32)

    eattr_pad = jnp.pad(edge_attrs, ((0, 0), (0, 16 - edge_attrs.shape[1])))
    rpe_w_pad = jnp.pad(params['rpe_W'], ((0, 16 - params['rpe_W'].shape[0]), (0, 0)))
    rpe_vecs = jnp.stack([params['rpe_b'], params['rpe_ln_g'], params['rpe_ln_b']])
    eas = [_rpe_call(eattr_pad[i * EP:(i + 1) * EP], rpe_w_pad, rpe_vecs, EP)
           for i in range(P)]

    x = lane_feats
    for p in params['layers']:
        w_dst = jnp.concatenate([p['mem_W'][:D], p['Wq']], axis=1)
        w_src = p['mem_W'][D:2 * D]
        w3 = p['mem_W'][2 * D:]
        wkv = jnp.concatenate([p['eu_W'], p['Wk'], p['Wv']], axis=1)
        evecs = jnp.stack([p['mem_b'], p['mem_ln_g'], p['mem_ln_b'],
                           p['eu_b'], p['eu_ln_g'], p['eu_ln_b'],
                           p['en_g'], p['en_b']])
        nvecs = jnp.stack([p['n1_g'], p['n1_b'], p['ffn_b2'],
                           p['n2_g'], p['n2_b'],
                           p['n1_g'], p['n1_g'], p['n1_g']])
        fb1 = jnp.broadcast_to(p['ffn_b1'][None, :], (8, 2 * D))

        t_dst, t_src = _node_pre_call(x, w_dst, w_src, N)
        wvs, wes = [], []
        for i in range(P):
            g1, g2 = _gather(t_dst, t_src, dsts[i], srcs[i], N, EP)
            eas[i], wv, we = _edge_call(g1, g2, eas[i], w3, wkv, evecs, EP)
            wvs.append(wv)
            wes.append(we)
        pv0 = pv1 = pe0 = pe1 = None
        for i in range(P):
            a0, a1 = _scatter_add(wvs[i], dsts[i], zeros_slab, N, EP)
            b0, b1 = _scatter_add(wes[i], dsts[i], zeros_slab, N, EP)
            pv0 = a0 if pv0 is None else pv0 + a0
            pv1 = a1 if pv1 is None else pv1 + a1
            pe0 = b0 if pe0 is None else pe0 + b0
            pe1 = b1 if pe1 is None else pe1 + b1
        x = _node_post_call(x, pv0, pv1, pe0, pe1, p['Wo'], p['ffn_W1'],
                            p['ffn_W2'], nvecs, fb1, N)
    return x
